# Initial kernel scaffold; baseline (speedup 1.0000x reference)
import jax
import jax.numpy as jnp
from jax.experimental import pallas as pl


def kernel(input, target):
    raise NotImplementedError("write your pallas kernel here")


# speedup vs baseline: 42.0822x; 42.0822x over previous
import functools

import jax
import jax.numpy as jnp
from jax import lax
from jax.experimental import pallas as pl
from jax.experimental.pallas import tpu as pltpu
from jax.experimental.pallas import tpu_sc as plsc

_B = 128
_V = 100000
_L = 16
_W = 2048
_GRID = -(-_V // _W)
_ROWS_PER_SUBCORE = 16
_ACTIVE_SUBCORES = _B // _ROWS_PER_SUBCORE

def _sc_gather(tab, target):
    nc = plsc.get_sparse_core_info().num_cores

    def body(tab_hbm, tgt_hbm, t_hbm, idx_v, tgt_v, t_v, sem):
        wid = lax.axis_index("s") * nc + lax.axis_index("c")

        @pl.when(wid < _ACTIVE_SUBCORES)
        def _():
            base = wid * _ROWS_PER_SUBCORE
            pltpu.sync_copy(tgt_hbm.at[pl.ds(base, _ROWS_PER_SUBCORE)], tgt_v)
            i = base + lax.iota(jnp.int32, _L)
            idx_v[...] = i * _V + tgt_v[...]
            pltpu.async_copy(tab_hbm.at[idx_v], t_v, sem).wait()
            pltpu.sync_copy(t_v, t_hbm.at[pl.ds(base, _ROWS_PER_SUBCORE)])

    mesh = plsc.VectorSubcoreMesh(core_axis_name="c", subcore_axis_name="s")
    k = pl.kernel(
        body,
        mesh=mesh,
        out_type=jax.ShapeDtypeStruct((_B,), jnp.float32),
        scratch_types=[
            pltpu.VMEM((_L,), jnp.int32),
            pltpu.VMEM((_L,), jnp.int32),
            pltpu.VMEM((_L,), jnp.float32),
            pltpu.SemaphoreType.DMA,
        ],
    )
    return k(tab, target)


def _count_body(tgt_ref, t_ref, x_ref, out_ref, acc_gt, acc_eq):
    b = pl.program_id(0)

    @pl.when(b == 0)
    def _():
        acc_gt[...] = jnp.zeros_like(acc_gt)
        acc_eq[...] = jnp.zeros_like(acc_eq)

    x = x_ref[...]
    t = t_ref[...]
    tgt = tgt_ref[...]
    col = b * _W + lax.broadcasted_iota(jnp.int32, (_B, _W), 1)
    gt = (x > t) & (col < _V)
    eq = (x == t) & (col < tgt)
    acc_gt[...] += jnp.sum(gt.astype(jnp.float32), axis=1, keepdims=True)
    acc_eq[...] += jnp.sum(eq.astype(jnp.float32), axis=1, keepdims=True)

    @pl.when(b == _GRID - 1)
    def _():
        rank = acc_gt[...] + acc_eq[...]
        apk = ((rank == 0.0).astype(jnp.float32)
               + (rank == 1.0).astype(jnp.float32) * 0.5
               + (rank == 2.0).astype(jnp.float32) * (1.0 / 3.0))
        out_ref[...] = jnp.sum(apk, axis=(0, 1), keepdims=True) * (1.0 / _B)


def _tc_count(x, t, target):
    return pl.pallas_call(
        _count_body,
        grid=(_GRID,),
        in_specs=[
            pl.BlockSpec((_B, 1), lambda b: (0, 0)),
            pl.BlockSpec((_B, 1), lambda b: (0, 0)),
            pl.BlockSpec((_B, _W), lambda b: (0, b)),
        ],
        out_specs=pl.BlockSpec((1, 1), lambda b: (0, 0)),
        out_shape=jax.ShapeDtypeStruct((1, 1), jnp.float32),
        scratch_shapes=[
            pltpu.VMEM((_B, 1), jnp.float32),
            pltpu.VMEM((_B, 1), jnp.float32),
        ],
        compiler_params=pltpu.CompilerParams(
            dimension_semantics=("arbitrary",),
        ),
    )(target, t, x)


def kernel(input, target):
    tab = input.reshape(_B * _V)
    t = _sc_gather(tab, target)
    res = _tc_count(input, t.reshape(_B, 1), target.reshape(_B, 1))
    return res[0, 0]


# speedup vs baseline: 69.6095x; 1.6541x over previous
import functools

import jax
import jax.numpy as jnp
from jax import lax
from jax.experimental import pallas as pl
from jax.experimental.pallas import tpu as pltpu
from jax.experimental.pallas import tpu_sc as plsc

_B = 128
_V = 100000
_L = 16
_W = 2048
_GRID = -(-_V // _W)
_ROWS_PER_SUBCORE = 16
_ACTIVE_SUBCORES = _B // _ROWS_PER_SUBCORE

def _sc_gather(tab, target):
    nc = plsc.get_sparse_core_info().num_cores

    def body(tab_hbm, tgt_hbm, t_hbm, tgt_v, pad_v, t_v, sem):
        wid = lax.axis_index("s") * nc + lax.axis_index("c")

        @pl.when(wid < _ACTIVE_SUBCORES)
        def _():
            base = wid * _ROWS_PER_SUBCORE
            pltpu.sync_copy(tgt_hbm.at[pl.ds(base, _ROWS_PER_SUBCORE)], tgt_v)
            tv = tgt_v[...]
            copies = []
            for r in range(_ROWS_PER_SUBCORE):
                c_al = (tv[r] // 8) * 8
                copies.append(pltpu.async_copy(
                    tab_hbm.at[base + r, pl.ds(c_al, 8)],
                    pad_v.at[pl.ds(r * 8, 8)], sem))
            for c in copies:
                c.wait()
            idx = lax.iota(jnp.int32, _L) * 8 + lax.bitwise_and(tgt_v[...], 7)
            t_v[...] = plsc.load_gather(pad_v, [idx])
            pltpu.sync_copy(t_v, t_hbm.at[pl.ds(base, _ROWS_PER_SUBCORE)])

    mesh = plsc.VectorSubcoreMesh(core_axis_name="c", subcore_axis_name="s")
    k = pl.kernel(
        body,
        mesh=mesh,
        out_type=jax.ShapeDtypeStruct((_B,), jnp.float32),
        scratch_types=[
            pltpu.VMEM((_L,), jnp.int32),
            pltpu.VMEM((_L * 8,), jnp.float32),
            pltpu.VMEM((_L,), jnp.float32),
            pltpu.SemaphoreType.DMA,
        ],
        compiler_params=pltpu.CompilerParams(
            use_tc_tiling_on_sc=True, needs_layout_passes=False),
    )
    return k(tab, target)


def _count_body(tgt_ref, t_ref, x_ref, out_ref, acc_gt, acc_eq):
    b = pl.program_id(0)

    @pl.when(b == 0)
    def _():
        acc_gt[...] = jnp.zeros_like(acc_gt)
        acc_eq[...] = jnp.zeros_like(acc_eq)

    x = x_ref[...]
    t = t_ref[...]
    tgt = tgt_ref[...]
    col = b * _W + lax.broadcasted_iota(jnp.int32, (_B, _W), 1)
    gt = (x > t) & (col < _V)
    eq = (x == t) & (col < tgt)
    acc_gt[...] += jnp.sum(gt.astype(jnp.float32), axis=1, keepdims=True)
    acc_eq[...] += jnp.sum(eq.astype(jnp.float32), axis=1, keepdims=True)

    @pl.when(b == _GRID - 1)
    def _():
        rank = acc_gt[...] + acc_eq[...]
        apk = ((rank == 0.0).astype(jnp.float32)
               + (rank == 1.0).astype(jnp.float32) * 0.5
               + (rank == 2.0).astype(jnp.float32) * (1.0 / 3.0))
        out_ref[...] = jnp.sum(apk, axis=(0, 1), keepdims=True) * (1.0 / _B)


def _tc_count(x, t, target):
    return pl.pallas_call(
        _count_body,
        grid=(_GRID,),
        in_specs=[
            pl.BlockSpec((_B, 1), lambda b: (0, 0)),
            pl.BlockSpec((_B, 1), lambda b: (0, 0)),
            pl.BlockSpec((_B, _W), lambda b: (0, b)),
        ],
        out_specs=pl.BlockSpec((1, 1), lambda b: (0, 0)),
        out_shape=jax.ShapeDtypeStruct((1, 1), jnp.float32),
        scratch_shapes=[
            pltpu.VMEM((_B, 1), jnp.float32),
            pltpu.VMEM((_B, 1), jnp.float32),
        ],
        compiler_params=pltpu.CompilerParams(
            dimension_semantics=("arbitrary",),
        ),
    )(target, t, x)


def kernel(input, target):
    t = _sc_gather(input, target)
    res = _tc_count(input, t.reshape(_B, 1), target.reshape(_B, 1))
    return res[0, 0]


# speedup vs baseline: 82.9809x; 1.1921x over previous
import functools

import jax
import jax.numpy as jnp
from jax import lax
from jax.experimental import pallas as pl
from jax.experimental.pallas import tpu as pltpu
from jax.experimental.pallas import tpu_sc as plsc

_B = 128
_V = 100000
_L = 16
_RB = 8
_ROWS_PER_SUBCORE = 16
_ACTIVE_SUBCORES = _B // _ROWS_PER_SUBCORE

def _sc_gather(tab, target):
    nc = plsc.get_sparse_core_info().num_cores

    def body(tab_hbm, tgt_hbm, t_hbm, tgt_v, pad_v, t_v, sem):
        wid = lax.axis_index("s") * nc + lax.axis_index("c")

        @pl.when(wid < _ACTIVE_SUBCORES)
        def _():
            base = wid * _ROWS_PER_SUBCORE
            pltpu.sync_copy(tgt_hbm.at[pl.ds(base, _ROWS_PER_SUBCORE)], tgt_v)
            tv = tgt_v[...]
            copies = []
            for r in range(_ROWS_PER_SUBCORE):
                c_al = (tv[r] // 8) * 8
                copies.append(pltpu.async_copy(
                    tab_hbm.at[base + r, pl.ds(c_al, 8)],
                    pad_v.at[pl.ds(r * 8, 8)], sem))
            for c in copies:
                c.wait()
            idx = lax.iota(jnp.int32, _L) * 8 + lax.bitwise_and(tgt_v[...], 7)
            t_v[...] = plsc.load_gather(pad_v, [idx])
            pltpu.sync_copy(t_v, t_hbm.at[pl.ds(base, _ROWS_PER_SUBCORE)])

    mesh = plsc.VectorSubcoreMesh(core_axis_name="c", subcore_axis_name="s")
    k = pl.kernel(
        body,
        mesh=mesh,
        out_type=jax.ShapeDtypeStruct((_B,), jnp.float32),
        scratch_types=[
            pltpu.VMEM((_L,), jnp.int32),
            pltpu.VMEM((_L * 8,), jnp.float32),
            pltpu.VMEM((_L,), jnp.float32),
            pltpu.SemaphoreType.DMA,
        ],
        compiler_params=pltpu.CompilerParams(
            use_tc_tiling_on_sc=True, needs_layout_passes=False),
    )
    return k(tab, target)


def _count_body(tgt_ref, t_ref, x_ref, out_ref):
    b = pl.program_id(0)
    x = x_ref[...]
    t = t_ref[...]
    tgt = tgt_ref[...]
    col = lax.broadcasted_iota(jnp.int32, (_RB, _V), 1)
    pred = (x > t) | ((x == t) & (col < tgt))
    rank = jnp.sum(pred.astype(jnp.float32), axis=1, keepdims=True)
    apk = ((rank == 0.0).astype(jnp.float32)
           + (rank == 1.0).astype(jnp.float32) * 0.5
           + (rank == 2.0).astype(jnp.float32) * (1.0 / 3.0))
    part = jnp.sum(apk, axis=(0, 1), keepdims=True) * (1.0 / _B)

    @pl.when(b == 0)
    def _():
        out_ref[...] = part

    @pl.when(b != 0)
    def _():
        out_ref[...] += part


def _tc_count(x, t, target):
    grid = _B // _RB
    return pl.pallas_call(
        _count_body,
        grid=(grid,),
        in_specs=[
            pl.BlockSpec((_RB, 1), lambda b: (b, 0)),
            pl.BlockSpec((_RB, 1), lambda b: (b, 0)),
            pl.BlockSpec((_RB, _V), lambda b: (b, 0)),
        ],
        out_specs=pl.BlockSpec((1, 1), lambda b: (0, 0)),
        out_shape=jax.ShapeDtypeStruct((1, 1), jnp.float32),
        compiler_params=pltpu.CompilerParams(
            dimension_semantics=("arbitrary",),
        ),
    )(target, t, x)


def kernel(input, target):
    t = _sc_gather(input, target)
    res = _tc_count(input, t.reshape(_B, 1), target.reshape(_B, 1))
    return res[0, 0]


# speedup vs baseline: 88.5062x; 1.0666x over previous
import functools

import jax
import jax.numpy as jnp
from jax import lax
from jax.experimental import pallas as pl
from jax.experimental.pallas import tpu as pltpu
from jax.experimental.pallas import tpu_sc as plsc

_B = 128
_V = 100000
_L = 16
_RB = 8
_ROWS_PER_SUBCORE = 16
_ACTIVE_SUBCORES = _B // _ROWS_PER_SUBCORE

def _sc_gather(tab, target):
    nc = plsc.get_sparse_core_info().num_cores

    def body(tab_hbm, tgt_hbm, t_hbm, tgt_v, pad_v, t_v, sem):
        wid = lax.axis_index("s") * nc + lax.axis_index("c")

        @pl.when(wid < _ACTIVE_SUBCORES)
        def _():
            base = wid * _ROWS_PER_SUBCORE
            pltpu.sync_copy(tgt_hbm.at[pl.ds(base, _ROWS_PER_SUBCORE)], tgt_v)
            tv = tgt_v[...]
            copies = []
            for r in range(_ROWS_PER_SUBCORE):
                c_al = (tv[r] // 8) * 8
                copies.append(pltpu.async_copy(
                    tab_hbm.at[base + r, pl.ds(c_al, 8)],
                    pad_v.at[pl.ds(r * 8, 8)], sem))
            for c in copies:
                c.wait()
            idx = lax.iota(jnp.int32, _L) * 8 + lax.bitwise_and(tgt_v[...], 7)
            t_v[...] = plsc.load_gather(pad_v, [idx])
            pltpu.sync_copy(t_v, t_hbm.at[pl.ds(base, _ROWS_PER_SUBCORE)])

    mesh = plsc.VectorSubcoreMesh(core_axis_name="c", subcore_axis_name="s")
    k = pl.kernel(
        body,
        mesh=mesh,
        out_type=jax.ShapeDtypeStruct((_B,), jnp.float32),
        scratch_types=[
            pltpu.VMEM((_L,), jnp.int32),
            pltpu.VMEM((_L * 8,), jnp.float32),
            pltpu.VMEM((_L,), jnp.float32),
            pltpu.SemaphoreType.DMA,
        ],
        compiler_params=pltpu.CompilerParams(
            use_tc_tiling_on_sc=True, needs_layout_passes=False),
    )
    return k(tab, target)


_NSPLIT = 4


def _count_body(tgt_ref, t_ref, *refs):
    b = pl.program_id(0)
    x_refs, out_ref = refs[:_NSPLIT], refs[_NSPLIT]
    t = t_ref[...]
    tgt = tgt_ref[...]
    col = lax.broadcasted_iota(jnp.int32, (_RB, _V), 1)
    part = None
    for p in range(_NSPLIT):
        x = x_refs[p][...]
        tp = t[p * _RB:(p + 1) * _RB]
        tgtp = tgt[p * _RB:(p + 1) * _RB]
        pred = (x > tp) | ((x == tp) & (col < tgtp))
        rank = jnp.sum(pred.astype(jnp.float32), axis=1, keepdims=True)
        apk = ((rank == 0.0).astype(jnp.float32)
               + (rank == 1.0).astype(jnp.float32) * 0.5
               + (rank == 2.0).astype(jnp.float32) * (1.0 / 3.0))
        s = jnp.sum(apk, axis=(0, 1), keepdims=True)
        part = s if part is None else part + s
    part = part * (1.0 / _B)

    @pl.when(b == 0)
    def _():
        out_ref[...] = part

    @pl.when(b != 0)
    def _():
        out_ref[...] += part


def _tc_count(x, t, target):
    rows_per_step = _RB * _NSPLIT
    grid = _B // rows_per_step
    x_specs = [
        pl.BlockSpec((_RB, _V), lambda b, p=p: (_NSPLIT * b + p, 0))
        for p in range(_NSPLIT)
    ]
    return pl.pallas_call(
        _count_body,
        grid=(grid,),
        in_specs=[
            pl.BlockSpec((rows_per_step, 1), lambda b: (b, 0)),
            pl.BlockSpec((rows_per_step, 1), lambda b: (b, 0)),
            *x_specs,
        ],
        out_specs=pl.BlockSpec((1, 1), lambda b: (0, 0)),
        out_shape=jax.ShapeDtypeStruct((1, 1), jnp.float32),
        compiler_params=pltpu.CompilerParams(
            dimension_semantics=("arbitrary",),
            vmem_limit_bytes=100 * 1024 * 1024,
        ),
    )(target, t, *([x] * _NSPLIT))


def kernel(input, target):
    t = _sc_gather(input, target)
    res = _tc_count(input, t.reshape(_B, 1), target.reshape(_B, 1))
    return res[0, 0]
